# baseline (speedup 1.0000x reference)
import functools

import jax
import jax.numpy as jnp
from jax import lax
from jax.experimental import pallas as pl
from jax.experimental.pallas import tpu as pltpu
from jax.experimental.pallas import tpu_sc as plsc

_VOCAB = 100000
_D = 64
_B = 16384
_CTX = 20
_NEG = 100
_TOT = _CTX + _NEG
_TOTP = 128
_LE_LAMBDA = 0.01

_NC = 2
_NS = 16
_NW = _NC * _NS
_BPW = _B // _NW
_CH = 128
_CHH = _CH // 2
_NCHUNK = _BPW // _CH
_NBUF = 8


def _sc_dots(in_w, out_w, inl, all_flat):
    mesh = plsc.VectorSubcoreMesh(core_axis_name="c", subcore_axis_name="s")

    @functools.partial(
        pl.kernel,
        mesh=mesh,
        out_type=jax.ShapeDtypeStruct((_B // _CHH, _CHH, _TOT), jnp.float32),
        scratch_types=[
            pltpu.VMEM((_BPW,), jnp.int32),
            pltpu.VMEM((_CHH, _D), jnp.float32),
            pltpu.VMEM((_CHH, _D), jnp.float32),
            pltpu.VMEM((_CH * _TOT,), jnp.int32),
            pltpu.VMEM((_CH * _TOT,), jnp.int32),
            pltpu.VMEM((_NBUF, _TOTP, _D), jnp.float32),
            pltpu.VMEM((_CHH, _TOTP), jnp.float32),
            pltpu.VMEM((_CHH, _TOTP), jnp.float32),
            pltpu.SemaphoreType.DMA((_NBUF,)),
            pltpu.SemaphoreType.DMA((2,)),
            pltpu.SemaphoreType.DMA((2,)),
            pltpu.SemaphoreType.DMA((2,)),
        ],
        compiler_params=pltpu.CompilerParams(
            needs_layout_passes=False, use_tc_tiling_on_sc=False
        ),
    )
    def k(in_w_hbm, out_w_hbm, inl_hbm, all_hbm, dots_hbm,
          uidx_v, ubuf_a, ubuf_b, lbl_a, lbl_b, rows_v, dots_a, dots_b,
          sems, usems, lsems, dsems):
        wid = lax.axis_index("s") * _NC + lax.axis_index("c")
        lane = lax.broadcasted_iota(jnp.int32, (16,), 0)
        ubufs = (ubuf_a, ubuf_b)
        lbufs = (lbl_a, lbl_b)
        dbufs = (dots_a, dots_b)

        def u_start(ci, h):
            pltpu.make_async_copy(
                in_w_hbm.at[uidx_v.at[pl.ds(ci * _CH + h * _CHH, _CHH)]],
                ubufs[h],
                usems.at[h],
            ).start()

        def u_wait(h):
            pltpu.make_async_copy(
                in_w_hbm.at[uidx_v.at[pl.ds(0, _CHH)]], ubufs[h], usems.at[h]
            ).wait()

        def lbl_start(ci, c2):
            pltpu.make_async_copy(
                all_hbm.at[
                    pl.ds((wid * _NCHUNK + ci) * (_CH * _TOT), _CH * _TOT)
                ],
                lbufs[c2],
                lsems.at[c2],
            ).start()

        def lbl_wait(c2):
            pltpu.make_async_copy(
                all_hbm.at[pl.ds(0, _CH * _TOT)], lbufs[c2], lsems.at[c2]
            ).wait()

        def dots_start(ci, h):
            pltpu.make_async_copy(
                dbufs[h].at[:, pl.ds(0, _TOT)],
                dots_hbm.at[wid * _NCHUNK * 2 + ci * 2 + h],
                dsems.at[h],
            ).start()

        def dots_wait(h):
            pltpu.make_async_copy(
                dbufs[h].at[:, pl.ds(0, _TOT)], dots_hbm.at[0], dsems.at[h]
            ).wait()

        pltpu.sync_copy(inl_hbm.at[pl.ds(wid * _BPW, _BPW)], uidx_v)
        lbl_start(0, 0)
        lbl_start(1, 1)
        u_start(0, 0)
        u_start(0, 1)

        def start_gather(lbl_v, b, slot):
            pltpu.make_async_copy(
                out_w_hbm.at[lbl_v.at[pl.ds(b * _TOT, _TOT)]],
                rows_v.at[slot].at[pl.ds(0, _TOT)],
                sems.at[slot],
            ).start()

        def wait_gather(slot):
            pltpu.make_async_copy(
                out_w_hbm.at[lbl_a.at[pl.ds(0, _TOT)]],
                rows_v.at[slot].at[pl.ds(0, _TOT)],
                sems.at[slot],
            ).wait()

        def compute(b, slot, ubuf, dots_v):
            buf = rows_v.at[slot]
            u0 = ubuf[b, pl.ds(0, 16)]
            u1 = ubuf[b, pl.ds(16, 16)]
            u2 = ubuf[b, pl.ds(32, 16)]
            u3 = ubuf[b, pl.ds(48, 16)]

            def gbody(g, _):
                def cbody(cc, d):
                    c = g * 16 + cc
                    p0 = buf[c, pl.ds(0, 16)] * u0
                    p1 = buf[c, pl.ds(16, 16)] * u1
                    p2 = buf[c, pl.ds(32, 16)] * u2
                    p3 = buf[c, pl.ds(48, 16)] * u3
                    p = (p0 + p1) + (p2 + p3)
                    return jnp.where(lane == cc, jnp.sum(p), d)

                d = lax.fori_loop(
                    0, 16, cbody, jnp.zeros((16,), jnp.float32), unroll=4
                )
                dots_v[b, pl.ds(g * 16, 16)] = d
                return 0

            lax.fori_loop(0, _TOTP // 16, gbody, 0)

        def cp_body(cp, _):
            for c2 in range(2):
                ci = cp * 2 + c2
                lbl_wait(c2)
                for s in range(_NBUF - 1):
                    start_gather(lbufs[c2], s, s)

                for h in range(2):
                    u_wait(h)

                    @pl.when(ci >= 1)
                    def _():
                        dots_wait(h)

                    def quad_body(i4, _):
                        for kk in range(_NBUF):
                            bh = i4 * _NBUF + kk
                            bc = h * _CHH + bh
                            nb = bc + (_NBUF - 1)

                            @pl.when(nb < _CH)
                            def _():
                                start_gather(
                                    lbufs[c2], nb, (kk + _NBUF - 1) % _NBUF
                                )

                            wait_gather(kk)
                            compute(bh, kk, ubufs[h], dbufs[h])
                        return 0

                    lax.fori_loop(0, _CHH // _NBUF, quad_body, 0)

                    dots_start(ci, h)

                    @pl.when(ci + 1 < _NCHUNK)
                    def _():
                        u_start(ci + 1, h)

                @pl.when(ci + 2 < _NCHUNK)
                def _():
                    lbl_start(ci + 2, c2)
            return 0

        lax.fori_loop(0, _NCHUNK // 2, cp_body, 0)
        dots_wait(0)
        dots_wait(1)

    return k(in_w, out_w, inl, all_flat)


def _tc_body(dots_ref, pi_ref, pj_ref, out_ref, le_ref):
    d = dots_ref[...]
    pos = d[:, : _CTX]
    neg = -d[:, _CTX:]

    def ls(x):
        return jnp.minimum(x, 0.0) - jnp.log1p(jnp.exp(-jnp.abs(x)))

    total = jnp.sum(ls(pos)) + jnp.sum(ls(neg))
    loss_graph = -total / _B

    diff = pi_ref[...] - pj_ref[...]
    nrm = jnp.sqrt(jnp.sum(diff * diff, axis=1))
    l2 = jnp.sum(nrm)
    le = 0.5 * l2 * l2 * _LE_LAMBDA

    out_ref[...] = jnp.reshape(loss_graph + le, (1, 1))
    le_ref[...] = jnp.reshape(le, (1, 1))


def _tc_reduce(dots, pair_i, pair_j):
    return pl.pallas_call(
        _tc_body,
        out_shape=(
            jax.ShapeDtypeStruct((1, 1), jnp.float32),
            jax.ShapeDtypeStruct((1, 1), jnp.float32),
        ),
    )(dots, pair_i, pair_j)


def kernel(input_labels, pos_labels, neg_labels, in_embed_w, out_embed_w):
    inl = input_labels.astype(jnp.int32)
    all_lbl = jnp.concatenate([pos_labels, neg_labels], axis=1).astype(jnp.int32)
    all_flat = all_lbl.reshape(-1)

    dots3 = _sc_dots(in_embed_w, out_embed_w, inl, all_flat)
    dots = dots3.reshape(_B, _TOT)

    first = in_embed_w[: 2 * 32].reshape(32, 2, _D)
    pair_i = first[:, 0, :]
    pair_j = first[:, 1, :]

    loss_combined, loss_le = _tc_reduce(dots, pair_i, pair_j)
    return (loss_combined[0, 0], loss_le[0, 0])


# speedup vs baseline: 1.0339x; 1.0339x over previous
import functools

import jax
import jax.numpy as jnp
from jax import lax
from jax.experimental import pallas as pl
from jax.experimental.pallas import tpu as pltpu
from jax.experimental.pallas import tpu_sc as plsc

_VOCAB = 100000
_D = 64
_B = 16384
_CTX = 20
_NEG = 100
_TOT = _CTX + _NEG
_TOTP = 128
_LE_LAMBDA = 0.01

_NC = 2
_NS = 16
_NW = _NC * _NS
_BPW = _B // _NW
_CH = 128
_CHH = _CH // 2
_NCHUNK = _BPW // _CH
_NBUF = 8


def _sc_dots(in_w, out_w, inl, all_flat):
    mesh = plsc.VectorSubcoreMesh(core_axis_name="c", subcore_axis_name="s")

    @functools.partial(
        pl.kernel,
        mesh=mesh,
        out_type=jax.ShapeDtypeStruct((_B // _CHH, _CHH, _TOTP), jnp.float32),
        scratch_types=[
            pltpu.VMEM((_BPW,), jnp.int32),
            pltpu.VMEM((_CHH, _D), jnp.float32),
            pltpu.VMEM((_CHH, _D), jnp.float32),
            pltpu.VMEM((_CH * _TOT,), jnp.int32),
            pltpu.VMEM((_CH * _TOT,), jnp.int32),
            pltpu.VMEM((_NBUF, _TOTP, _D), jnp.float32),
            pltpu.VMEM((_CHH, _TOTP), jnp.float32),
            pltpu.VMEM((_CHH, _TOTP), jnp.float32),
            pltpu.SemaphoreType.DMA((_NBUF,)),
            pltpu.SemaphoreType.DMA((2,)),
            pltpu.SemaphoreType.DMA((2,)),
            pltpu.SemaphoreType.DMA((2,)),
        ],
        compiler_params=pltpu.CompilerParams(
            needs_layout_passes=False, use_tc_tiling_on_sc=False
        ),
    )
    def k(in_w_hbm, out_w_hbm, inl_hbm, all_hbm, dots_hbm,
          uidx_v, ubuf_a, ubuf_b, lbl_a, lbl_b, rows_v, dots_a, dots_b,
          sems, usems, lsems, dsems):
        wid = lax.axis_index("s") * _NC + lax.axis_index("c")
        lane = lax.broadcasted_iota(jnp.int32, (16,), 0)
        ubufs = (ubuf_a, ubuf_b)
        lbufs = (lbl_a, lbl_b)
        dbufs = (dots_a, dots_b)

        def u_start(ci, h):
            pltpu.make_async_copy(
                in_w_hbm.at[uidx_v.at[pl.ds(ci * _CH + h * _CHH, _CHH)]],
                ubufs[h],
                usems.at[h],
            ).start()

        def u_wait(h):
            pltpu.make_async_copy(
                in_w_hbm.at[uidx_v.at[pl.ds(0, _CHH)]], ubufs[h], usems.at[h]
            ).wait()

        def lbl_start(ci, c2):
            pltpu.make_async_copy(
                all_hbm.at[
                    pl.ds((wid * _NCHUNK + ci) * (_CH * _TOT), _CH * _TOT)
                ],
                lbufs[c2],
                lsems.at[c2],
            ).start()

        def lbl_wait(c2):
            pltpu.make_async_copy(
                all_hbm.at[pl.ds(0, _CH * _TOT)], lbufs[c2], lsems.at[c2]
            ).wait()

        def dots_start(ci, h):
            pltpu.make_async_copy(
                dbufs[h], dots_hbm.at[wid * _NCHUNK * 2 + ci * 2 + h], dsems.at[h]
            ).start()

        def dots_wait(h):
            pltpu.make_async_copy(
                dbufs[h], dots_hbm.at[0], dsems.at[h]
            ).wait()

        pltpu.sync_copy(inl_hbm.at[pl.ds(wid * _BPW, _BPW)], uidx_v)
        lbl_start(0, 0)
        lbl_start(1, 1)
        u_start(0, 0)
        u_start(0, 1)

        def start_gather(lbl_v, b, slot):
            pltpu.make_async_copy(
                out_w_hbm.at[lbl_v.at[pl.ds(b * _TOT, _TOT)]],
                rows_v.at[slot].at[pl.ds(0, _TOT)],
                sems.at[slot],
            ).start()

        def wait_gather(slot):
            pltpu.make_async_copy(
                out_w_hbm.at[lbl_a.at[pl.ds(0, _TOT)]],
                rows_v.at[slot].at[pl.ds(0, _TOT)],
                sems.at[slot],
            ).wait()

        def compute(b, slot, ubuf, dots_v):
            buf = rows_v.at[slot]
            u0 = ubuf[b, pl.ds(0, 16)]
            u1 = ubuf[b, pl.ds(16, 16)]
            u2 = ubuf[b, pl.ds(32, 16)]
            u3 = ubuf[b, pl.ds(48, 16)]

            def gbody(g, _):
                def cbody(cc, d):
                    c = g * 16 + cc
                    p0 = buf[c, pl.ds(0, 16)] * u0
                    p1 = buf[c, pl.ds(16, 16)] * u1
                    p2 = buf[c, pl.ds(32, 16)] * u2
                    p3 = buf[c, pl.ds(48, 16)] * u3
                    p = (p0 + p1) + (p2 + p3)
                    return jnp.where(lane == cc, jnp.sum(p), d)

                d = lax.fori_loop(
                    0, 16, cbody, jnp.zeros((16,), jnp.float32), unroll=4
                )
                dots_v[b, pl.ds(g * 16, 16)] = d
                return 0

            lax.fori_loop(0, _TOTP // 16, gbody, 0)

        def cp_body(cp, _):
            for c2 in range(2):
                ci = cp * 2 + c2
                lbl_wait(c2)
                for s in range(_NBUF - 1):
                    start_gather(lbufs[c2], s, s)

                for h in range(2):
                    u_wait(h)

                    @pl.when(ci >= 1)
                    def _():
                        dots_wait(h)

                    def quad_body(i4, _):
                        for kk in range(_NBUF):
                            bh = i4 * _NBUF + kk
                            bc = h * _CHH + bh
                            nb = bc + (_NBUF - 1)

                            @pl.when(nb < _CH)
                            def _():
                                start_gather(
                                    lbufs[c2], nb, (kk + _NBUF - 1) % _NBUF
                                )

                            wait_gather(kk)
                            compute(bh, kk, ubufs[h], dbufs[h])
                        return 0

                    lax.fori_loop(0, _CHH // _NBUF, quad_body, 0)

                    dots_start(ci, h)

                    @pl.when(ci + 1 < _NCHUNK)
                    def _():
                        u_start(ci + 1, h)

                @pl.when(ci + 2 < _NCHUNK)
                def _():
                    lbl_start(ci + 2, c2)
            return 0

        lax.fori_loop(0, _NCHUNK // 2, cp_body, 0)
        dots_wait(0)
        dots_wait(1)

    return k(in_w, out_w, inl, all_flat)


def _tc_body(dots_ref, pi_ref, pj_ref, out_ref, le_ref):
    d = dots_ref[...]
    pos = d[:, : _CTX]
    neg = -d[:, _CTX:]

    def ls(x):
        return jnp.minimum(x, 0.0) - jnp.log1p(jnp.exp(-jnp.abs(x)))

    total = jnp.sum(ls(pos)) + jnp.sum(ls(neg))
    loss_graph = -total / _B

    diff = pi_ref[...] - pj_ref[...]
    nrm = jnp.sqrt(jnp.sum(diff * diff, axis=1))
    l2 = jnp.sum(nrm)
    le = 0.5 * l2 * l2 * _LE_LAMBDA

    out_ref[...] = jnp.reshape(loss_graph + le, (1, 1))
    le_ref[...] = jnp.reshape(le, (1, 1))


def _tc_reduce(dots, pair_i, pair_j):
    return pl.pallas_call(
        _tc_body,
        out_shape=(
            jax.ShapeDtypeStruct((1, 1), jnp.float32),
            jax.ShapeDtypeStruct((1, 1), jnp.float32),
        ),
    )(dots, pair_i, pair_j)


def kernel(input_labels, pos_labels, neg_labels, in_embed_w, out_embed_w):
    inl = input_labels.astype(jnp.int32)
    all_lbl = jnp.concatenate([pos_labels, neg_labels], axis=1).astype(jnp.int32)
    all_flat = all_lbl.reshape(-1)

    dots3 = _sc_dots(in_embed_w, out_embed_w, inl, all_flat)
    dots = dots3.reshape(_B, _TOTP)[:, :_TOT]

    first = in_embed_w[: 2 * 32].reshape(32, 2, _D)
    pair_i = first[:, 0, :]
    pair_j = first[:, 1, :]

    loss_combined, loss_le = _tc_reduce(dots, pair_i, pair_j)
    return (loss_combined[0, 0], loss_le[0, 0])


# speedup vs baseline: 1.0362x; 1.0023x over previous
import functools

import jax
import jax.numpy as jnp
from jax import lax
from jax.experimental import pallas as pl
from jax.experimental.pallas import tpu as pltpu
from jax.experimental.pallas import tpu_sc as plsc

_VOCAB = 100000
_D = 64
_B = 16384
_CTX = 20
_NEG = 100
_TOT = _CTX + _NEG
_TOTP = 128
_LE_LAMBDA = 0.01

_NC = 2
_NS = 16
_NW = _NC * _NS
_BPW = _B // _NW
_CH = 128
_CHH = _CH // 2
_NCHUNK = _BPW // _CH
_NBUF = 8


def _sc_dots(in_w, out_w, inl, all_flat):
    mesh = plsc.VectorSubcoreMesh(core_axis_name="c", subcore_axis_name="s")

    @functools.partial(
        pl.kernel,
        mesh=mesh,
        out_type=jax.ShapeDtypeStruct((_B // _CHH, _CHH, _TOTP), jnp.float32),
        scratch_types=[
            pltpu.VMEM((_BPW,), jnp.int32),
            pltpu.VMEM((_CHH, _D), jnp.float32),
            pltpu.VMEM((_CHH, _D), jnp.float32),
            pltpu.VMEM((_CH * _TOT,), jnp.int32),
            pltpu.VMEM((_CH * _TOT,), jnp.int32),
            pltpu.VMEM((_NBUF, _TOTP, _D), jnp.float32),
            pltpu.VMEM((_CHH, _TOTP), jnp.float32),
            pltpu.VMEM((_CHH, _TOTP), jnp.float32),
            pltpu.SemaphoreType.DMA((_NBUF,)),
            pltpu.SemaphoreType.DMA((2,)),
            pltpu.SemaphoreType.DMA((2,)),
            pltpu.SemaphoreType.DMA((2,)),
        ],
        compiler_params=pltpu.CompilerParams(
            needs_layout_passes=False, use_tc_tiling_on_sc=False
        ),
    )
    def k(in_w_hbm, out_w_hbm, inl_hbm, all_hbm, dots_hbm,
          uidx_v, ubuf_a, ubuf_b, lbl_a, lbl_b, rows_v, dots_a, dots_b,
          sems, usems, lsems, dsems):
        wid = lax.axis_index("s") * _NC + lax.axis_index("c")
        lane = lax.broadcasted_iota(jnp.int32, (16,), 0)
        ubufs = (ubuf_a, ubuf_b)
        lbufs = (lbl_a, lbl_b)
        dbufs = (dots_a, dots_b)

        def u_start(ci, h):
            pltpu.make_async_copy(
                in_w_hbm.at[uidx_v.at[pl.ds(ci * _CH + h * _CHH, _CHH)]],
                ubufs[h],
                usems.at[h],
            ).start()

        def u_wait(h):
            pltpu.make_async_copy(
                in_w_hbm.at[uidx_v.at[pl.ds(0, _CHH)]], ubufs[h], usems.at[h]
            ).wait()

        def lbl_start(ci, c2):
            pltpu.make_async_copy(
                all_hbm.at[
                    pl.ds((wid * _NCHUNK + ci) * (_CH * _TOT), _CH * _TOT)
                ],
                lbufs[c2],
                lsems.at[c2],
            ).start()

        def lbl_wait(c2):
            pltpu.make_async_copy(
                all_hbm.at[pl.ds(0, _CH * _TOT)], lbufs[c2], lsems.at[c2]
            ).wait()

        def dots_start(ci, h):
            pltpu.make_async_copy(
                dbufs[h], dots_hbm.at[wid * _NCHUNK * 2 + ci * 2 + h], dsems.at[h]
            ).start()

        def dots_wait(h):
            pltpu.make_async_copy(
                dbufs[h], dots_hbm.at[0], dsems.at[h]
            ).wait()

        pltpu.sync_copy(inl_hbm.at[pl.ds(wid * _BPW, _BPW)], uidx_v)
        lbl_start(0, 0)
        lbl_start(1, 1)
        u_start(0, 0)
        u_start(0, 1)

        def start_gather(lbl_v, b, slot):
            pltpu.make_async_copy(
                out_w_hbm.at[lbl_v.at[pl.ds(b * _TOT, _TOT)]],
                rows_v.at[slot].at[pl.ds(0, _TOT)],
                sems.at[slot],
            ).start()

        def wait_gather(slot):
            pltpu.make_async_copy(
                out_w_hbm.at[lbl_a.at[pl.ds(0, _TOT)]],
                rows_v.at[slot].at[pl.ds(0, _TOT)],
                sems.at[slot],
            ).wait()

        def compute(b, slot, ubuf, dots_v):
            buf = rows_v.at[slot]
            u0 = ubuf[b, pl.ds(0, 16)]
            u1 = ubuf[b, pl.ds(16, 16)]
            u2 = ubuf[b, pl.ds(32, 16)]
            u3 = ubuf[b, pl.ds(48, 16)]

            def gbody(g, _):
                def cbody(cc, d):
                    c = g * 16 + cc
                    p0 = buf[c, pl.ds(0, 16)] * u0
                    p1 = buf[c, pl.ds(16, 16)] * u1
                    p2 = buf[c, pl.ds(32, 16)] * u2
                    p3 = buf[c, pl.ds(48, 16)] * u3
                    p = (p0 + p1) + (p2 + p3)
                    return jnp.where(lane == cc, jnp.sum(p), d)

                d = lax.fori_loop(
                    0, 16, cbody, jnp.zeros((16,), jnp.float32), unroll=4
                )
                dots_v[b, pl.ds(g * 16, 16)] = d
                return 0

            lax.fori_loop(0, _TOTP // 16, gbody, 0)

        def cp_body(cp, _):
            for c2 in range(2):
                ci = cp * 2 + c2
                lbl_wait(c2)
                for s in range(_NBUF - 1):
                    start_gather(lbufs[c2], s, s)

                for h in range(2):
                    u_wait(h)

                    @pl.when(ci >= 1)
                    def _():
                        dots_wait(h)

                    def quad_body(i4, _):
                        for kk in range(_NBUF):
                            bh = i4 * _NBUF + kk
                            bc = h * _CHH + bh
                            nb = bc + (_NBUF - 1)

                            @pl.when(nb < _CH)
                            def _():
                                start_gather(
                                    lbufs[c2], nb, (kk + _NBUF - 1) % _NBUF
                                )

                            wait_gather(kk)
                            compute(bh, kk, ubufs[h], dbufs[h])
                        return 0

                    lax.fori_loop(0, _CHH // _NBUF, quad_body, 0)

                    dots_start(ci, h)

                    @pl.when(ci + 1 < _NCHUNK)
                    def _():
                        u_start(ci + 1, h)

                @pl.when(ci + 2 < _NCHUNK)
                def _():
                    lbl_start(ci + 2, c2)
            return 0

        lax.fori_loop(0, _NCHUNK // 2, cp_body, 0)
        dots_wait(0)
        dots_wait(1)

    return k(in_w, out_w, inl, all_flat)


def _tc_body(dots_ref, pi_ref, pj_ref, out_ref, le_ref):
    d = dots_ref[...]
    pos = d[:, : _CTX]
    neg = -d[:, _CTX:]

    def ls(x):
        return jnp.minimum(x, 0.0) - jnp.log1p(jnp.exp(-jnp.abs(x)))

    ncol = jax.lax.broadcasted_iota(jnp.int32, neg.shape, 1)
    neg_contrib = jnp.where(ncol < _NEG, ls(neg), 0.0)
    total = jnp.sum(ls(pos)) + jnp.sum(neg_contrib)
    loss_graph = -total / _B

    diff = pi_ref[...] - pj_ref[...]
    nrm = jnp.sqrt(jnp.sum(diff * diff, axis=1))
    l2 = jnp.sum(nrm)
    le = 0.5 * l2 * l2 * _LE_LAMBDA

    out_ref[...] = jnp.reshape(loss_graph + le, (1, 1))
    le_ref[...] = jnp.reshape(le, (1, 1))


def _tc_reduce(dots, pair_i, pair_j):
    return pl.pallas_call(
        _tc_body,
        out_shape=(
            jax.ShapeDtypeStruct((1, 1), jnp.float32),
            jax.ShapeDtypeStruct((1, 1), jnp.float32),
        ),
    )(dots, pair_i, pair_j)


def kernel(input_labels, pos_labels, neg_labels, in_embed_w, out_embed_w):
    inl = input_labels.astype(jnp.int32)
    all_lbl = jnp.concatenate([pos_labels, neg_labels], axis=1).astype(jnp.int32)
    all_flat = all_lbl.reshape(-1)

    dots3 = _sc_dots(in_embed_w, out_embed_w, inl, all_flat)
    dots = dots3.reshape(_B, _TOTP)

    first = in_embed_w[: 2 * 32].reshape(32, 2, _D)
    pair_i = first[:, 0, :]
    pair_j = first[:, 1, :]

    loss_combined, loss_le = _tc_reduce(dots, pair_i, pair_j)
    return (loss_combined[0, 0], loss_le[0, 0])
